# baseline (speedup 1.0000x reference)
import functools

import jax
import jax.numpy as jnp
from jax import lax
from jax.experimental import pallas as pl
from jax.experimental.pallas import tpu as pltpu
from jax.experimental.pallas import tpu_sc as plsc

EPS = 1e-06

NC = 2
NS = 16
NW = NC * NS

CHUNK = 125
G = 64
OUT = 3


def _sc_mesh():
    return plsc.VectorSubcoreMesh(core_axis_name="c", subcore_axis_name="s",
                                  num_cores=NC, num_subcores=NS)


_ROW_CHUNKS = [(k * 80, 80) for k in range(7)] + [(560, 72)]

CH = 80
CPB = 8
NBLK = 16
CPT = NBLK * CPB
EPT = CPT * CH


def _sc_agg_kernel(npad, d):
    rows_per_tile = npad // NS

    out_type = [jax.ShapeDtypeStruct((NC, npad, d), jnp.float32)]
    scratch = [
        pltpu.VMEM((2, CPB, CH), jnp.int32),
        pltpu.VMEM((2, CPB, CH), jnp.int32),
        pltpu.VMEM((2, CH, d), jnp.float32),
        pltpu.VMEM_SHARED((npad, d), jnp.float32),
        pltpu.SemaphoreType.DMA,
        pltpu.SemaphoreType.DMA,
        pltpu.SemaphoreType.DMA,
        pltpu.SemaphoreType.DMA,
    ]

    def body(t_hbm, src_hbm, dst_hbm, out_hbm,
             sidx, didx, rows, acc_sh, gsem, ssem, sisem, disem):
        zero16 = jnp.zeros((16,), jnp.float32)
        cid = lax.axis_index("c")
        sid = lax.axis_index("s")
        wid = cid * NS + sid
        base = sid * rows_per_tile

        def fill_zrow(i, c):
            for k in range(d // 16):
                rows[0, i, pl.ds(k * 16, 16)] = zero16
            return c
        lax.fori_loop(0, CH, fill_zrow, 0)
        for off, ln in _ROW_CHUNKS:
            pltpu.sync_copy(rows.at[0, pl.ds(0, ln)],
                            acc_sh.at[pl.ds(base + off, ln)])
        plsc.subcore_barrier()

        pltpu.sync_copy(src_hbm.at[wid, pl.ds(0, CPB)], sidx.at[0])
        pltpu.sync_copy(dst_hbm.at[wid, pl.ds(0, CPB)], didx.at[0])
        pltpu.async_copy(t_hbm.at[sidx.at[0, 0]], rows.at[0], gsem)

        def block_step(b, carry):
            bb = b % 2
            nb = (b + 1) % 2

            @pl.when(b + 1 < NBLK)
            def _prefetch_idx():
                off = pl.multiple_of((b + 1) * CPB, 8)
                pltpu.async_copy(src_hbm.at[wid, pl.ds(off, CPB)],
                                 sidx.at[nb], sisem)
                pltpu.async_copy(dst_hbm.at[wid, pl.ds(off, CPB)],
                                 didx.at[nb], disem)

            for k in range(CPB):
                par = k % 2
                pltpu.make_async_copy(t_hbm.at[pl.ds(0, CH)],
                                      rows.at[par], gsem).wait()
                pltpu.async_copy(rows.at[par], acc_sh.at[didx.at[bb, k]],
                                 ssem, add=True)
                if k > 0:
                    pltpu.make_async_copy(t_hbm.at[pl.ds(0, CH)],
                                          rows.at[1 - par], ssem).wait()
                else:
                    @pl.when(b > 0)
                    def _drain_prev():
                        pltpu.make_async_copy(t_hbm.at[pl.ds(0, CH)],
                                              rows.at[1 - par], ssem).wait()
                if k < CPB - 1:
                    pltpu.async_copy(t_hbm.at[sidx.at[bb, k + 1]],
                                     rows.at[1 - par], gsem)
                else:
                    @pl.when(b + 1 < NBLK)
                    def _next_gather():
                        pltpu.make_async_copy(
                            src_hbm.at[wid, pl.ds(0, CPB)],
                            sidx.at[nb], sisem).wait()
                        pltpu.make_async_copy(
                            dst_hbm.at[wid, pl.ds(0, CPB)],
                            didx.at[nb], disem).wait()
                        pltpu.async_copy(t_hbm.at[sidx.at[nb, 0]],
                                         rows.at[1 - par], gsem)
            return carry
        lax.fori_loop(0, NBLK, block_step, 0)
        pltpu.make_async_copy(t_hbm.at[pl.ds(0, CH)],
                              rows.at[(CPB - 1) % 2], ssem).wait()
        plsc.subcore_barrier()

        for off, ln in _ROW_CHUNKS:
            sl = pl.ds(base + off, ln)
            pltpu.sync_copy(acc_sh.at[sl], rows.at[0, pl.ds(0, ln)])
            pltpu.sync_copy(rows.at[0, pl.ds(0, ln)], out_hbm.at[cid].at[sl])

    return pl.kernel(body, out_type=out_type, mesh=_sc_mesh(),
                     scratch_types=scratch)


def _sc_count_kernel(npad, epw):
    groups = epw // 16
    out_type = [jax.ShapeDtypeStruct((NW, 1, npad), jnp.float32)]
    scratch = [
        pltpu.VMEM((groups, 16), jnp.int32),
        pltpu.VMEM((npad,), jnp.float32),
    ]

    def body(dst_hbm, cntout_hbm, dst_v, hist_v):
        zero16 = jnp.zeros((16,), jnp.float32)
        one16 = jnp.ones((16,), jnp.float32)
        cid = lax.axis_index("c")
        sid = lax.axis_index("s")
        wid = cid * NS + sid

        pltpu.sync_copy(dst_hbm.at[wid], dst_v)

        def zero_hist(k, c):
            hist_v[pl.ds(k * 16, 16)] = zero16
            return c
        lax.fori_loop(0, npad // 16, zero_hist, 0)

        def count_step(i, c):
            dvec = dst_v[i, :]
            plsc.addupdate_scatter(hist_v, [dvec], one16)
            return c
        lax.fori_loop(0, groups, count_step, 0)

        pltpu.sync_copy(hist_v, cntout_hbm.at[wid, 0])

    return pl.kernel(
        body, out_type=out_type, mesh=_sc_mesh(), scratch_types=scratch,
        compiler_params=pltpu.CompilerParams(needs_layout_passes=False))


def _lin_sp_body(x_ref, wt_ref, b_ref, o_ref):
    h = jnp.dot(x_ref[...], wt_ref[...],
                preferred_element_type=jnp.float32) + b_ref[...]
    o_ref[...] = h + EPS * jnp.sign(h)


def _mid_body(acc_ref, cnt_ref, wt_ref, b_ref, o_ref):
    a = acc_ref[0] + acc_ref[1]
    c = jnp.sum(cnt_ref[...], axis=1, keepdims=True)
    pooled = a / jnp.maximum(c, 1.0)
    m = pooled + EPS * jnp.sign(pooled)
    r = jnp.maximum(m, 0.0)
    h = jnp.dot(r, wt_ref[...], preferred_element_type=jnp.float32) + b_ref[...]
    o_ref[...] = h + EPS * jnp.sign(h)


def _final_body(acc_ref, cnt_ref, batch_ref, wt_ref, b_ref, o_ref,
                gsum_s, gcnt_s, *, nblocks, rowb):
    i = pl.program_id(0)

    @pl.when(i == 0)
    def _init():
        gsum_s[...] = jnp.zeros_like(gsum_s)
        gcnt_s[...] = jnp.zeros_like(gcnt_s)

    a = acc_ref[0] + acc_ref[1]
    c = jnp.sum(cnt_ref[...], axis=1, keepdims=True)
    pooled = a / jnp.maximum(c, 1.0)
    h = pooled + EPS * jnp.sign(pooled)

    bvals = jnp.broadcast_to(batch_ref[0], (G, rowb))
    gids = lax.broadcasted_iota(jnp.int32, (G, rowb), 0)
    onehot = (gids == bvals).astype(jnp.float32)
    gsum_s[...] += jnp.dot(onehot, h, preferred_element_type=jnp.float32)
    gcnt_s[...] += jnp.broadcast_to(
        jnp.sum(onehot, axis=1, keepdims=True), gcnt_s.shape)

    @pl.when(i == nblocks - 1)
    def _finish():
        graph = gsum_s[...] / jnp.maximum(gcnt_s[...], 1.0)
        o_ref[...] = jnp.dot(graph, wt_ref[...],
                             preferred_element_type=jnp.float32) + b_ref[...]


def kernel(x, edge_index, batch, W1, b1, W2, b2, W_out, b_out):
    n, d = x.shape
    e = edge_index.shape[1]
    rpt = 632
    npad = NS * rpt
    nblocks = 16
    rowb = rpt
    lin_rowb = n // 10
    assert n % 10 == 0 and npad >= n

    e_pad = NW * EPT
    pad_n = e_pad - e
    fake_dst = n + (jnp.arange(pad_n, dtype=jnp.int32) % (npad - n))
    src3 = jnp.concatenate(
        [edge_index[0], jnp.zeros((pad_n,), jnp.int32)]).reshape(NW, CPT, CH)
    dst3 = jnp.concatenate(
        [edge_index[1], fake_dst]).reshape(NW, CPT, CH)
    batch3 = jnp.pad(batch, (0, npad - n), constant_values=G).reshape(
        nblocks, 1, rowb)
    w1t = W1.T
    w2t = W2.T
    b1r = b1.reshape(1, d)
    b2r = b2.reshape(1, d)
    wot = jnp.pad(W_out.T, ((0, 0), (0, d - OUT)))
    bor = jnp.pad(b_out, (0, d - OUT)).reshape(1, d)

    lin1 = pl.pallas_call(
        _lin_sp_body,
        grid=(10,),
        in_specs=[pl.BlockSpec((lin_rowb, d), lambda i: (i, 0)),
                  pl.BlockSpec((d, d), lambda i: (0, 0)),
                  pl.BlockSpec((1, d), lambda i: (0, 0))],
        out_specs=pl.BlockSpec((lin_rowb, d), lambda i: (i, 0)),
        out_shape=jax.ShapeDtypeStruct((n, d), jnp.float32),
    )
    t1 = lin1(x, w1t, b1r)

    epw = e // NW
    dstc = edge_index[1].reshape(NW, epw // 16, 16)
    count_k = _sc_count_kernel(npad, epw)
    (cnt_raw,) = count_k(dstc)
    cnt1 = cnt_raw.reshape(NW, npad).T
    agg1 = _sc_agg_kernel(npad, d)
    (acc1,) = agg1(t1, src3, dst3)

    mid = pl.pallas_call(
        _mid_body,
        grid=(nblocks,),
        in_specs=[pl.BlockSpec((NC, rowb, d), lambda i: (0, i, 0)),
                  pl.BlockSpec((rowb, NW), lambda i: (i, 0)),
                  pl.BlockSpec((d, d), lambda i: (0, 0)),
                  pl.BlockSpec((1, d), lambda i: (0, 0))],
        out_specs=pl.BlockSpec((rowb, d), lambda i: (i, 0)),
        out_shape=jax.ShapeDtypeStruct((npad, d), jnp.float32),
    )
    t2 = mid(acc1, cnt1, w2t, b2r)

    agg2 = _sc_agg_kernel(npad, d)
    (acc2,) = agg2(t2, src3, dst3)

    final = pl.pallas_call(
        functools.partial(_final_body, nblocks=nblocks, rowb=rowb),
        grid=(nblocks,),
        in_specs=[pl.BlockSpec((NC, rowb, d), lambda i: (0, i, 0)),
                  pl.BlockSpec((rowb, NW), lambda i: (i, 0)),
                  pl.BlockSpec((1, 1, rowb), lambda i: (i, 0, 0)),
                  pl.BlockSpec((d, d), lambda i: (0, 0)),
                  pl.BlockSpec((1, d), lambda i: (0, 0))],
        out_specs=pl.BlockSpec((G, d), lambda i: (0, 0)),
        out_shape=jax.ShapeDtypeStruct((G, d), jnp.float32),
        scratch_shapes=[pltpu.VMEM((G, d), jnp.float32),
                        pltpu.VMEM((G, d), jnp.float32)],
    )
    logits_pad = final(acc2, cnt1, batch3, wot, bor)
    return logits_pad[:, :OUT]


# speedup vs baseline: 2.5528x; 2.5528x over previous
import functools

import jax
import jax.numpy as jnp
from jax import lax
from jax.experimental import pallas as pl
from jax.experimental.pallas import tpu as pltpu
from jax.experimental.pallas import tpu_sc as plsc

EPS = 1e-06

NC = 2
NS = 16
NW = NC * NS

CHUNK = 125
G = 64
OUT = 3


def _sc_mesh():
    return plsc.VectorSubcoreMesh(core_axis_name="c", subcore_axis_name="s",
                                  num_cores=NC, num_subcores=NS)


_ROW_CHUNKS = [(k * 120, 120) for k in range(5)] + [(600, 32)]


def _sc_agg_kernel(npad, d, num_chunks):
    rows_per_tile = npad // NS
    half = num_chunks // 2

    out_type = [jax.ShapeDtypeStruct((NC, npad, d), jnp.float32)]
    scratch = [
        pltpu.VMEM((half, CHUNK), jnp.int32),
        pltpu.VMEM((num_chunks, CHUNK), jnp.int32),
        pltpu.VMEM((CHUNK, d), jnp.float32),
        pltpu.VMEM_SHARED((npad, d), jnp.float32),
    ]

    def body(t_hbm, src_hbm, dst_hbm, out_hbm, src_v, dst_v, rows_v, acc_sh):
        zero16 = jnp.zeros((16,), jnp.float32)
        cid = lax.axis_index("c")
        sid = lax.axis_index("s")
        wid = cid * NS + sid

        pltpu.sync_copy(dst_hbm.at[wid], dst_v)

        def fill_zrow(i, c):
            for k in range(d // 16):
                rows_v[i, pl.ds(k * 16, 16)] = zero16
            return c
        lax.fori_loop(0, CHUNK, fill_zrow, 0)

        base = sid * rows_per_tile
        for off, ln in _ROW_CHUNKS:
            pltpu.sync_copy(rows_v.at[pl.ds(0, ln)],
                            acc_sh.at[pl.ds(base + off, ln)])
        plsc.subcore_barrier()

        def chunk_step(j, joff):
            pltpu.sync_copy(t_hbm.at[src_v.at[j]], rows_v)
            pltpu.sync_copy(rows_v, acc_sh.at[dst_v.at[j + joff]], add=True)
            return joff
        for phase in range(2):
            pltpu.sync_copy(src_hbm.at[wid, pl.ds(phase * half, half)], src_v)
            lax.fori_loop(0, half, chunk_step, phase * half)
        plsc.subcore_barrier()

        for off, ln in _ROW_CHUNKS:
            sl = pl.ds(base + off, ln)
            pltpu.sync_copy(acc_sh.at[sl], rows_v.at[pl.ds(0, ln)])
            pltpu.sync_copy(rows_v.at[pl.ds(0, ln)], out_hbm.at[cid].at[sl])

    return pl.kernel(body, out_type=out_type, mesh=_sc_mesh(),
                     scratch_types=scratch)


def _sc_count_kernel(npad, epw):
    groups = epw // 16
    out_type = [jax.ShapeDtypeStruct((NW, 1, npad), jnp.float32)]
    scratch = [
        pltpu.VMEM((groups, 16), jnp.int32),
        pltpu.VMEM((npad,), jnp.float32),
    ]

    def body(dst_hbm, cntout_hbm, dst_v, hist_v):
        zero16 = jnp.zeros((16,), jnp.float32)
        one16 = jnp.ones((16,), jnp.float32)
        cid = lax.axis_index("c")
        sid = lax.axis_index("s")
        wid = cid * NS + sid

        pltpu.sync_copy(dst_hbm.at[wid], dst_v)

        def zero_hist(k, c):
            hist_v[pl.ds(k * 16, 16)] = zero16
            return c
        lax.fori_loop(0, npad // 16, zero_hist, 0)

        def count_step(i, c):
            dvec = dst_v[i, :]
            plsc.addupdate_scatter(hist_v, [dvec], one16)
            return c
        lax.fori_loop(0, groups, count_step, 0)

        pltpu.sync_copy(hist_v, cntout_hbm.at[wid, 0])

    return pl.kernel(
        body, out_type=out_type, mesh=_sc_mesh(), scratch_types=scratch,
        compiler_params=pltpu.CompilerParams(needs_layout_passes=False))


def _lin_sp_body(x_ref, wt_ref, b_ref, o_ref):
    h = jnp.dot(x_ref[...], wt_ref[...],
                preferred_element_type=jnp.float32) + b_ref[...]
    o_ref[...] = h + EPS * jnp.sign(h)


def _mid_body(acc_ref, cnt_ref, wt_ref, b_ref, o_ref):
    a = acc_ref[0] + acc_ref[1]
    c = jnp.sum(cnt_ref[...], axis=1, keepdims=True)
    pooled = a / jnp.maximum(c, 1.0)
    m = pooled + EPS * jnp.sign(pooled)
    r = jnp.maximum(m, 0.0)
    h = jnp.dot(r, wt_ref[...], preferred_element_type=jnp.float32) + b_ref[...]
    o_ref[...] = h + EPS * jnp.sign(h)


def _final_body(acc_ref, cnt_ref, batch_ref, wt_ref, b_ref, o_ref,
                gsum_s, gcnt_s, *, nblocks, rowb):
    i = pl.program_id(0)

    @pl.when(i == 0)
    def _init():
        gsum_s[...] = jnp.zeros_like(gsum_s)
        gcnt_s[...] = jnp.zeros_like(gcnt_s)

    a = acc_ref[0] + acc_ref[1]
    c = jnp.sum(cnt_ref[...], axis=1, keepdims=True)
    pooled = a / jnp.maximum(c, 1.0)
    h = pooled + EPS * jnp.sign(pooled)

    bvals = jnp.broadcast_to(batch_ref[0], (G, rowb))
    gids = lax.broadcasted_iota(jnp.int32, (G, rowb), 0)
    onehot = (gids == bvals).astype(jnp.float32)
    gsum_s[...] += jnp.dot(onehot, h, preferred_element_type=jnp.float32)
    gcnt_s[...] += jnp.broadcast_to(
        jnp.sum(onehot, axis=1, keepdims=True), gcnt_s.shape)

    @pl.when(i == nblocks - 1)
    def _finish():
        graph = gsum_s[...] / jnp.maximum(gcnt_s[...], 1.0)
        o_ref[...] = jnp.dot(graph, wt_ref[...],
                             preferred_element_type=jnp.float32) + b_ref[...]


def kernel(x, edge_index, batch, W1, b1, W2, b2, W_out, b_out):
    n, d = x.shape
    e = edge_index.shape[1]
    rpt = 632
    npad = NS * rpt
    nblocks = 16
    rowb = rpt
    lin_rowb = n // 10
    num_chunks = e // (NW * CHUNK)
    assert e % (NW * CHUNK) == 0 and n % 10 == 0 and npad >= n

    src3 = edge_index[0].reshape(NW, num_chunks, CHUNK)
    dst3 = edge_index[1].reshape(NW, num_chunks, CHUNK)
    batch3 = jnp.pad(batch, (0, npad - n), constant_values=G).reshape(
        nblocks, 1, rowb)
    w1t = W1.T
    w2t = W2.T
    b1r = b1.reshape(1, d)
    b2r = b2.reshape(1, d)
    wot = jnp.pad(W_out.T, ((0, 0), (0, d - OUT)))
    bor = jnp.pad(b_out, (0, d - OUT)).reshape(1, d)

    lin1 = pl.pallas_call(
        _lin_sp_body,
        grid=(10,),
        in_specs=[pl.BlockSpec((lin_rowb, d), lambda i: (i, 0)),
                  pl.BlockSpec((d, d), lambda i: (0, 0)),
                  pl.BlockSpec((1, d), lambda i: (0, 0))],
        out_specs=pl.BlockSpec((lin_rowb, d), lambda i: (i, 0)),
        out_shape=jax.ShapeDtypeStruct((n, d), jnp.float32),
    )
    t1 = lin1(x, w1t, b1r)

    epw = e // NW
    dstc = edge_index[1].reshape(NW, epw // 16, 16)
    count_k = _sc_count_kernel(npad, epw)
    (cnt_raw,) = count_k(dstc)
    cnt1 = cnt_raw.reshape(NW, npad).T
    agg1 = _sc_agg_kernel(npad, d, num_chunks)
    (acc1,) = agg1(t1, src3, dst3)

    mid = pl.pallas_call(
        _mid_body,
        grid=(nblocks,),
        in_specs=[pl.BlockSpec((NC, rowb, d), lambda i: (0, i, 0)),
                  pl.BlockSpec((rowb, NW), lambda i: (i, 0)),
                  pl.BlockSpec((d, d), lambda i: (0, 0)),
                  pl.BlockSpec((1, d), lambda i: (0, 0))],
        out_specs=pl.BlockSpec((rowb, d), lambda i: (i, 0)),
        out_shape=jax.ShapeDtypeStruct((npad, d), jnp.float32),
    )
    t2 = mid(acc1, cnt1, w2t, b2r)

    agg2 = _sc_agg_kernel(npad, d, num_chunks)
    (acc2,) = agg2(t2, src3, dst3)

    final = pl.pallas_call(
        functools.partial(_final_body, nblocks=nblocks, rowb=rowb),
        grid=(nblocks,),
        in_specs=[pl.BlockSpec((NC, rowb, d), lambda i: (0, i, 0)),
                  pl.BlockSpec((rowb, NW), lambda i: (i, 0)),
                  pl.BlockSpec((1, 1, rowb), lambda i: (i, 0, 0)),
                  pl.BlockSpec((d, d), lambda i: (0, 0)),
                  pl.BlockSpec((1, d), lambda i: (0, 0))],
        out_specs=pl.BlockSpec((G, d), lambda i: (0, 0)),
        out_shape=jax.ShapeDtypeStruct((G, d), jnp.float32),
        scratch_shapes=[pltpu.VMEM((G, d), jnp.float32),
                        pltpu.VMEM((G, d), jnp.float32)],
    )
    logits_pad = final(acc2, cnt1, batch3, wot, bor)
    return logits_pad[:, :OUT]
